# Initial kernel scaffold; baseline (speedup 1.0000x reference)
import jax
import jax.numpy as jnp
from jax.experimental import pallas as pl


def kernel(x, edge_index, edge_attr, W1, a1_src, a1_dst, b1, W2, a2_src, a2_dst, b2, Wc, bc):
    raise NotImplementedError("write your pallas kernel here")


# speedup vs baseline: 39.1928x; 39.1928x over previous
import functools

import jax
import jax.numpy as jnp
from jax import lax
from jax.experimental import pallas as pl
from jax.experimental.pallas import tpu as pltpu
from jax.experimental.pallas import tpu_sc as plsc

N = 10000
N_PAD = 10240
D_IN = 128
D_H = 16
N_CLS = 10
E = 320000

NC = 2
NS = 16
NW = NC * NS
CW = 128
E_TOT = E + N
CHUNKS = -(-E_TOT // (NW * CW))
E_PAD = NW * CHUNKS * CW
ROWS_PER_SUB = N_PAD // NS

_f32 = jnp.float32


def _tc1_body(x_ref, w_ref, a2_ref, h_ref, st_ref):
    h = jnp.dot(x_ref[...], w_ref[...], preferred_element_type=_f32)
    h_ref[...] = h
    st_ref[...] = jnp.dot(h, a2_ref[...], preferred_element_type=_f32)


def _tc_dense1(x_pad, W1, a2):
    return pl.pallas_call(
        _tc1_body,
        out_shape=[
            jax.ShapeDtypeStruct((N_PAD, D_H), _f32),
            jax.ShapeDtypeStruct((N_PAD, 2), _f32),
        ],
    )(x_pad, W1, a2)


def _tc2_body(acc_ref, b_ref, w_ref, a2_ref, h_ref, st_ref):
    acc = acc_ref[0] + acc_ref[1]
    num = acc[:, 0:D_H]
    den = acc[:, D_H:D_H + 1]
    o = num / (den + 1e-16) + b_ref[...]
    o = jnp.maximum(o, 0.0)
    h = jnp.dot(o, w_ref[...], preferred_element_type=_f32)
    h_ref[...] = h
    st_ref[...] = jnp.dot(h, a2_ref[...], preferred_element_type=_f32)


def _tc_dense2(acc, b1, W2, a2):
    return pl.pallas_call(
        _tc2_body,
        out_shape=[
            jax.ShapeDtypeStruct((N_PAD, D_H), _f32),
            jax.ShapeDtypeStruct((N_PAD, 2), _f32),
        ],
    )(acc, b1, W2, a2)


def _tc3_body(acc_ref, b_ref, wc_ref, bc_ref, out_ref):
    acc = acc_ref[0] + acc_ref[1]
    o = acc[:, 0:D_H] / (acc[:, D_H:D_H + 1] + 1e-16) + b_ref[...]
    logits = jnp.dot(o, wc_ref[...], preferred_element_type=_f32) + bc_ref[...]
    m = jnp.max(logits, axis=1, keepdims=True)
    z = logits - m
    lse = jnp.log(jnp.sum(jnp.exp(z), axis=1, keepdims=True))
    out_ref[...] = z - lse


def _tc_dense3(acc, b2, Wc, bc):
    return pl.pallas_call(
        _tc3_body,
        out_shape=jax.ShapeDtypeStruct((N_PAD, N_CLS), _f32),
    )(acc, b2, Wc, bc)


def _sc_body(src_hbm, dst_hbm, st_hbm, h_hbm, zeros_hbm, acc_out,
             src_v, dst_v, st_v, rows_v, rows32_v, acc_sh):
    cid = lax.axis_index("c")
    sid = lax.axis_index("s")
    wid = sid * NC + cid

    pltpu.sync_copy(src_hbm.at[wid], src_v)
    pltpu.sync_copy(dst_hbm.at[wid], dst_v)
    pltpu.sync_copy(st_hbm, st_v)
    pltpu.sync_copy(zeros_hbm.at[pl.ds(sid * ROWS_PER_SUB, ROWS_PER_SUB)],
                    acc_sh.at[pl.ds(sid * ROWS_PER_SUB, ROWS_PER_SUB)])

    zeros16 = jnp.zeros((16,), jnp.int32)
    ones16 = jnp.ones((16,), jnp.int32)

    def cbody(j, carry):
        pltpu.sync_copy(h_hbm.at[src_v.at[j]], rows_v)
        for k in range(CW // 16):
            srcs = src_v[j, pl.ds(k * 16, 16)]
            dsts = dst_v[j, pl.ds(k * 16, 16)]
            sv = plsc.load_gather(st_v, [srcs, zeros16])
            tv = plsc.load_gather(st_v, [dsts, ones16])
            z = sv + tv
            z = jnp.where(z >= 0.0, z, 0.2 * z)
            wv = jnp.exp(z)
            for i in range(16):
                wsc = wv[i]
                r = k * 16 + i
                rows32_v[r, 0:D_H] = rows_v[r, :] * wsc
                rows32_v[r, D_H:2 * D_H] = jnp.full((16,), 1.0, _f32) * wsc
        pltpu.sync_copy(rows32_v, acc_sh.at[dst_v.at[j]], add=True)
        return carry

    lax.fori_loop(0, CHUNKS, cbody, 0)
    plsc.subcore_barrier()

    pltpu.sync_copy(acc_sh.at[pl.ds(sid * ROWS_PER_SUB, ROWS_PER_SUB)],
                    acc_out.at[cid].at[pl.ds(sid * ROWS_PER_SUB, ROWS_PER_SUB)])


_sc_layer = functools.partial(
    pl.kernel,
    out_type=jax.ShapeDtypeStruct((NC, N_PAD, 2 * D_H), _f32),
    mesh=plsc.VectorSubcoreMesh(core_axis_name="c", subcore_axis_name="s"),
    compiler_params=pltpu.CompilerParams(needs_layout_passes=False,
                                         use_tc_tiling_on_sc=False),
    scratch_types=[
        pltpu.VMEM((CHUNKS, CW), jnp.int32),
        pltpu.VMEM((CHUNKS, CW), jnp.int32),
        pltpu.VMEM((N_PAD, 2), _f32),
        pltpu.VMEM((CW, D_H), _f32),
        pltpu.VMEM((CW, 2 * D_H), _f32),
        pltpu.VMEM_SHARED((N_PAD, 2 * D_H), _f32),
    ],
)(_sc_body)


def kernel(x, edge_index, edge_attr, W1, a1_src, a1_dst, b1, W2, a2_src,
           a2_dst, b2, Wc, bc):
    del edge_attr
    src = edge_index[0].astype(jnp.int32)
    dst = edge_index[1].astype(jnp.int32)
    loop_idx = jnp.arange(N, dtype=jnp.int32)
    pad_idx = N + (jnp.arange(E_PAD - E_TOT, dtype=jnp.int32) % (N_PAD - N))
    src_all = jnp.concatenate([src, loop_idx, pad_idx]).reshape(NW, CHUNKS, CW)
    dst_all = jnp.concatenate([dst, loop_idx, pad_idx]).reshape(NW, CHUNKS, CW)

    x_pad = jnp.pad(x, ((0, N_PAD - N), (0, 0)))
    zeros = jnp.zeros((N_PAD, 2 * D_H), _f32)

    a1 = jnp.concatenate([a1_src[:, None], a1_dst[:, None]], axis=1)
    a2 = jnp.concatenate([a2_src[:, None], a2_dst[:, None]], axis=1)

    h1, st1 = _tc_dense1(x_pad, W1, a1)
    acc1 = _sc_layer(src_all, dst_all, st1, h1, zeros)
    h2, st2 = _tc_dense2(acc1, b1[None, :], W2, a2)
    acc2 = _sc_layer(src_all, dst_all, st2, h2, zeros)
    out = _tc_dense3(acc2, b2[None, :], Wc, bc[None, :])
    return out[:N]


# speedup vs baseline: 50.8976x; 1.2986x over previous
import functools

import jax
import jax.numpy as jnp
from jax import lax
from jax.experimental import pallas as pl
from jax.experimental.pallas import tpu as pltpu
from jax.experimental.pallas import tpu_sc as plsc

N = 10000
D_IN = 128
D_H = 16
N_CLS = 10
E = 320000

NC = 2
NS = 16
NW = NC * NS
CW = 128
E_TOT = E + N
CHUNKS = -(-E_TOT // (NW * CW))
E_PAD = NW * CHUNKS * CW
N_ACC = N + NS
ROWS_PER_SUB = N_ACC // NS

_f32 = jnp.float32


def _tc1_body(x_ref, w_ref, a2_ref, h_ref, st_ref):
    h = jnp.dot(x_ref[...], w_ref[...], preferred_element_type=_f32)
    h_ref[...] = h
    st_ref[...] = jnp.dot(h, a2_ref[...], preferred_element_type=_f32)


def _tc_dense1(x_pad, W1, a2):
    return pl.pallas_call(
        _tc1_body,
        out_shape=[
            jax.ShapeDtypeStruct((N_ACC, D_H), _f32),
            jax.ShapeDtypeStruct((N_ACC, 2), _f32),
        ],
    )(x_pad, W1, a2)


def _tc2_body(acc_ref, b_ref, w_ref, a2_ref, h_ref, st_ref):
    acc = acc_ref[0] + acc_ref[1]
    num = acc[:, 0:D_H]
    den = acc[:, D_H:D_H + 1]
    o = num / (den + 1e-16) + b_ref[...]
    o = jnp.maximum(o, 0.0)
    h = jnp.dot(o, w_ref[...], preferred_element_type=_f32)
    h_ref[...] = h
    st_ref[...] = jnp.dot(h, a2_ref[...], preferred_element_type=_f32)


def _tc_dense2(acc, b1, W2, a2):
    return pl.pallas_call(
        _tc2_body,
        out_shape=[
            jax.ShapeDtypeStruct((N_ACC, D_H), _f32),
            jax.ShapeDtypeStruct((N_ACC, 2), _f32),
        ],
    )(acc, b1, W2, a2)


def _tc3_body(acc_ref, b_ref, wc_ref, bc_ref, out_ref):
    acc = acc_ref[0] + acc_ref[1]
    o = acc[:, 0:D_H] / (acc[:, D_H:D_H + 1] + 1e-16) + b_ref[...]
    logits = jnp.dot(o, wc_ref[...], preferred_element_type=_f32) + bc_ref[...]
    m = jnp.max(logits, axis=1, keepdims=True)
    z = logits - m
    lse = jnp.log(jnp.sum(jnp.exp(z), axis=1, keepdims=True))
    out_ref[...] = z - lse


def _tc_dense3(acc, b2, Wc, bc):
    return pl.pallas_call(
        _tc3_body,
        out_shape=jax.ShapeDtypeStruct((N_ACC, N_CLS), _f32),
    )(acc, b2, Wc, bc)


def _sc_body(src_hbm, dst_hbm, st_hbm, h_hbm, zeros_hbm, acc_out,
             src_v, dst_v, st_v, rows_v, rows32_v, acc_sh, gsem0, gsem1):
    cid = lax.axis_index("c")
    sid = lax.axis_index("s")
    wid = sid * NC + cid

    pltpu.sync_copy(src_hbm.at[wid], src_v)
    pltpu.sync_copy(dst_hbm.at[wid], dst_v)
    pltpu.sync_copy(st_hbm, st_v)
    pltpu.sync_copy(zeros_hbm.at[pl.ds(sid * ROWS_PER_SUB, ROWS_PER_SUB)],
                    acc_sh.at[pl.ds(sid * ROWS_PER_SUB, ROWS_PER_SUB)])

    zeros16 = jnp.zeros((16,), jnp.int32)
    ones16 = jnp.ones((16,), jnp.int32)
    gsems = (gsem0, gsem1)

    pltpu.async_copy(h_hbm.at[src_v.at[0]], rows_v.at[0], gsem0)

    def _process(jj, b):
        pltpu.make_async_copy(h_hbm.at[src_v.at[jj]], rows_v.at[b],
                              gsems[b]).wait()
        nb = 1 - b

        @pl.when(jj < CHUNKS - 1)
        def _():
            pltpu.async_copy(h_hbm.at[src_v.at[jj + 1]], rows_v.at[nb],
                             gsems[nb])

        for k in range(CW // 16):
            srcs = src_v[jj, pl.ds(k * 16, 16)]
            dsts = dst_v[jj, pl.ds(k * 16, 16)]
            sv = plsc.load_gather(st_v, [srcs, zeros16])
            tv = plsc.load_gather(st_v, [dsts, ones16])
            z = sv + tv
            z = jnp.where(z >= 0.0, z, 0.2 * z)
            wv = jnp.exp(z)
            for i in range(16):
                wsc = wv[i]
                r = k * 16 + i
                rows32_v[r, 0:D_H] = rows_v[b, r, :] * wsc
                rows32_v[r, D_H:2 * D_H] = jnp.full((16,), 1.0, _f32) * wsc
        pltpu.sync_copy(rows32_v, acc_sh.at[dst_v.at[jj]], add=True)

    def cbody(p, carry):
        _process(2 * p, 0)
        _process(2 * p + 1, 1)
        return carry

    lax.fori_loop(0, CHUNKS // 2, cbody, 0)
    _process(CHUNKS - 1, 0)
    plsc.subcore_barrier()

    pltpu.sync_copy(acc_sh.at[pl.ds(sid * ROWS_PER_SUB, ROWS_PER_SUB)],
                    acc_out.at[cid].at[pl.ds(sid * ROWS_PER_SUB, ROWS_PER_SUB)])


_sc_layer = functools.partial(
    pl.kernel,
    out_type=jax.ShapeDtypeStruct((NC, N_ACC, 2 * D_H), _f32),
    mesh=plsc.VectorSubcoreMesh(core_axis_name="c", subcore_axis_name="s"),
    compiler_params=pltpu.CompilerParams(needs_layout_passes=False,
                                         use_tc_tiling_on_sc=False),
    scratch_types=[
        pltpu.VMEM((CHUNKS, CW), jnp.int32),
        pltpu.VMEM((CHUNKS, CW), jnp.int32),
        pltpu.VMEM((N_ACC, 2), _f32),
        pltpu.VMEM((2, CW, D_H), _f32),
        pltpu.VMEM((CW, 2 * D_H), _f32),
        pltpu.VMEM_SHARED((N_ACC, 2 * D_H), _f32),
        pltpu.SemaphoreType.DMA,
        pltpu.SemaphoreType.DMA,
    ],
)(_sc_body)


def kernel(x, edge_index, edge_attr, W1, a1_src, a1_dst, b1, W2, a2_src,
           a2_dst, b2, Wc, bc):
    del edge_attr
    src = edge_index[0].astype(jnp.int32)
    dst = edge_index[1].astype(jnp.int32)
    loop_idx = jnp.arange(N, dtype=jnp.int32)
    pad_idx = N + (jnp.arange(E_PAD - E_TOT, dtype=jnp.int32) % NS)
    src_all = jnp.concatenate([src, loop_idx, pad_idx]).reshape(NW, CHUNKS, CW)
    dst_all = jnp.concatenate([dst, loop_idx, pad_idx]).reshape(NW, CHUNKS, CW)

    x_pad = jnp.pad(x, ((0, N_ACC - N), (0, 0)))
    zeros = jnp.zeros((N_ACC, 2 * D_H), _f32)

    a1 = jnp.concatenate([a1_src[:, None], a1_dst[:, None]], axis=1)
    a2 = jnp.concatenate([a2_src[:, None], a2_dst[:, None]], axis=1)

    h1, st1 = _tc_dense1(x_pad, W1, a1)
    acc1 = _sc_layer(src_all, dst_all, st1, h1, zeros)
    h2, st2 = _tc_dense2(acc1, b1[None, :], W2, a2)
    acc2 = _sc_layer(src_all, dst_all, st2, h2, zeros)
    out = _tc_dense3(acc2, b2[None, :], Wc, bc[None, :])
    return out[:N]
